# Initial kernel scaffold; baseline (speedup 1.0000x reference)
import jax
import jax.numpy as jnp
from jax.experimental import pallas as pl


def kernel(x, edge_index, emb, W_gcn, b_gcn, gamma, beta, W_ih, W_hh, b_ih, b_hh, W_fc, b_fc):
    raise NotImplementedError("write your pallas kernel here")


# speedup vs baseline: 3.6713x; 3.6713x over previous
import jax
import jax.numpy as jnp
from jax import lax
from jax.experimental import pallas as pl
from jax.experimental.pallas import tpu as pltpu
from jax.experimental.pallas import tpu_sc as plsc

N = 50000
E = 800000
H = 96
HG = 32
NG = 3
NUM_STEPS = 32
VOCAB = 10
CHUNK = 128
NCHUNKS = E // CHUNK
NTILES = 32
ROWS_PAD = 50176
TROWS = ROWS_PAD // 16
ZROWS = 196
BN = 1000
GRID = N // BN

_MESH = plsc.VectorSubcoreMesh(
    core_axis_name="c", subcore_axis_name="s", num_cores=2, num_subcores=16)

_SC_PARAMS = pltpu.CompilerParams(use_tc_tiling_on_sc=False)

_f32 = jnp.float32


def _zero_vmem_2d(ref, rows, cols):
    z = jnp.zeros((16,), _f32)

    def body(i, carry):
        for k in range(cols // 16):
            ref[i, pl.ds(k * 16, 16)] = z
        return carry

    lax.fori_loop(0, rows, body, 0)


def _deg_body(dst_hbm, q0, q1, dstv, ones_b, zb, acc, _):
    c = lax.axis_index("c")
    s = lax.axis_index("s")
    w = c * 16 + s

    one = jnp.full((16,), 1.0, _f32)

    def fill(i, carry):
        ones_b[i, pl.ds(0, 16)] = one
        return carry

    lax.fori_loop(0, CHUNK, fill, 0)
    _zero_vmem_2d(zb, ZROWS, 16)

    for k in range(TROWS // ZROWS):
        pltpu.sync_copy(zb, acc.at[pl.ds(s * TROWS + k * ZROWS, ZROWS)])
    plsc.subcore_barrier()

    nch = (NCHUNKS - w + NTILES - 1) // NTILES

    def chunk(j, carry):
        base = (w + j * NTILES) * CHUNK
        pltpu.sync_copy(dst_hbm.at[pl.ds(base, CHUNK)], dstv)
        pltpu.sync_copy(ones_b, acc.at[dstv], add=True)
        return carry

    lax.fori_loop(0, nch, chunk, 0)
    plsc.subcore_barrier()

    @pl.when(c == 0)
    def _():
        pltpu.sync_copy(acc.at[pl.ds(s * TROWS, TROWS)],
                        q0.at[pl.ds(s * TROWS, TROWS)])

    @pl.when(c == 1)
    def _():
        pltpu.sync_copy(acc.at[pl.ds(s * TROWS, TROWS)],
                        q1.at[pl.ds(s * TROWS, TROWS)])


_deg_call = pl.kernel(
    _deg_body,
    out_type=(jax.ShapeDtypeStruct((ROWS_PAD, 16), _f32),
              jax.ShapeDtypeStruct((ROWS_PAD, 16), _f32)),
    mesh=_MESH,
    scratch_types=[
        pltpu.VMEM((CHUNK,), jnp.int32),
        pltpu.VMEM((CHUNK, 16), _f32),
        pltpu.VMEM((ZROWS, 16), _f32),
        pltpu.VMEM_SHARED((ROWS_PAD, 16), _f32),
        pltpu.SemaphoreType.DMA,
    ],
    compiler_params=_SC_PARAMS,
)


def _agg_body(y0, y1, y2, src_hbm, dst_hbm,
              p00, p01, p02, p10, p11, p12,
              srcv, dstv, gbuf, zb, acc, _):
    c = lax.axis_index("c")
    s = lax.axis_index("s")
    w = c * 16 + s

    _zero_vmem_2d(zb, ZROWS, HG)

    ys = (y0, y1, y2)
    outs0 = (p00, p01, p02)
    outs1 = (p10, p11, p12)
    nch = (NCHUNKS - w + NTILES - 1) // NTILES

    for g in range(NG):
        yg = ys[g]

        for k in range(TROWS // ZROWS):
            pltpu.sync_copy(zb, acc.at[pl.ds(s * TROWS + k * ZROWS, ZROWS)])
        plsc.subcore_barrier()

        def chunk(j, carry):
            base = (w + j * NTILES) * CHUNK
            pltpu.sync_copy(src_hbm.at[pl.ds(base, CHUNK)], srcv)
            pltpu.sync_copy(dst_hbm.at[pl.ds(base, CHUNK)], dstv)
            pltpu.sync_copy(yg.at[srcv], gbuf)
            pltpu.sync_copy(gbuf, acc.at[dstv], add=True)
            return carry

        lax.fori_loop(0, nch, chunk, 0)
        plsc.subcore_barrier()

        @pl.when(c == 0)
        def _():
            pltpu.sync_copy(acc.at[pl.ds(s * TROWS, TROWS)],
                            outs0[g].at[pl.ds(s * TROWS, TROWS)])

        @pl.when(c == 1)
        def _():
            pltpu.sync_copy(acc.at[pl.ds(s * TROWS, TROWS)],
                            outs1[g].at[pl.ds(s * TROWS, TROWS)])

        plsc.subcore_barrier()


_agg_call = pl.kernel(
    _agg_body,
    out_type=tuple(jax.ShapeDtypeStruct((ROWS_PAD, HG), _f32)
                   for _ in range(6)),
    mesh=_MESH,
    scratch_types=[
        pltpu.VMEM((CHUNK,), jnp.int32),
        pltpu.VMEM((CHUNK,), jnp.int32),
        pltpu.VMEM((CHUNK, HG), _f32),
        pltpu.VMEM((ZROWS, HG), _f32),
        pltpu.VMEM_SHARED((ROWS_PAD, HG), _f32),
        pltpu.SemaphoreType.DMA,
    ],
    compiler_params=_SC_PARAMS,
)


def _mm_t(a, b):
    return lax.dot_general(a.astype(jnp.bfloat16), b.astype(jnp.bfloat16),
                           (((1,), (1,)), ((), ())),
                           preferred_element_type=_f32)


def _init_body(x_ref, emb_ref, q0_ref, q1_ref, wg_ref,
               h_ref, y0_ref, y1_ref, y2_ref, dis_ref):
    deg = q0_ref[:, 0:1] + q1_ref[:, 0:1] + 1.0
    dis = lax.rsqrt(deg)
    xi = x_ref[:, :]
    h0 = jnp.zeros((xi.shape[0], H), _f32)
    for v in range(VOCAB):
        h0 = h0 + jnp.where(xi == v, 1.0, 0.0) * emb_ref[v:v + 1, :]
    y = dis * _mm_t(h0, wg_ref[:, :])
    h_ref[:, :] = h0
    y0_ref[:, :] = y[:, 0:HG]
    y1_ref[:, :] = y[:, HG:2 * HG]
    y2_ref[:, :] = y[:, 2 * HG:H]
    dis_ref[:, :] = dis


def _upd_body(p00, p01, p02, p10, p11, p12, y0, y1, y2, h_ref, dis_ref,
              wg, wir, wiz, win, whr, whz, whn,
              bg, bir, biz, bin_, bhr, bhz, bhn, gma, bta,
              hn_ref, z0_ref, z1_ref, z2_ref):
    ssum = jnp.concatenate(
        [p00[:, :] + p10[:, :], p01[:, :] + p11[:, :], p02[:, :] + p12[:, :]],
        axis=1)
    y = jnp.concatenate([y0[:, :], y1[:, :], y2[:, :]], axis=1)
    d = dis_ref[:, :]
    m = d * (ssum + y) + bg[:, :]
    m = jnp.maximum(m, 0.0)
    mu = jnp.mean(m, axis=1, keepdims=True)
    va = jnp.mean((m - mu) ** 2, axis=1, keepdims=True)
    mh = (m - mu) * lax.rsqrt(va + 1e-5) * gma[:, :] + bta[:, :]
    hh = h_ref[:, :]
    ir = _mm_t(mh, wir[:, :]) + bir[:, :]
    iz = _mm_t(mh, wiz[:, :]) + biz[:, :]
    inn = _mm_t(mh, win[:, :]) + bin_[:, :]
    hr = _mm_t(hh, whr[:, :]) + bhr[:, :]
    hz = _mm_t(hh, whz[:, :]) + bhz[:, :]
    hn = _mm_t(hh, whn[:, :]) + bhn[:, :]
    r = jax.nn.sigmoid(ir + hr)
    z = jax.nn.sigmoid(iz + hz)
    n = jnp.tanh(inn + r * hn)
    hnew = (1.0 - z) * n + z * hh
    hn_ref[:, :] = hnew
    yn = d * _mm_t(hnew, wg[:, :])
    z0_ref[:, :] = yn[:, 0:HG]
    z1_ref[:, :] = yn[:, HG:2 * HG]
    z2_ref[:, :] = yn[:, 2 * HG:H]


def _fin_body(h_ref, wfc_ref, bfc_ref, o_ref):
    o_ref[:, :] = _mm_t(h_ref[:, :], wfc_ref[:, :]) + bfc_ref[:, :]


def _bs(shape):
    return pl.BlockSpec(shape, lambda i: (i, 0))


def _full(a):
    return pl.BlockSpec(a.shape, lambda i: (0, 0))


def kernel(x, edge_index, emb, W_gcn, b_gcn, gamma, beta,
           W_ih, W_hh, b_ih, b_hh, W_fc, b_fc):
    src = edge_index[0].astype(jnp.int32)
    dst = edge_index[1].astype(jnp.int32)
    x2 = x.astype(jnp.int32).reshape(N, 1)

    wir, wiz, win = W_ih[:H], W_ih[H:2 * H], W_ih[2 * H:]
    whr, whz, whn = W_hh[:H], W_hh[H:2 * H], W_hh[2 * H:]
    bir, biz, bin_ = (b_ih[:H].reshape(1, H), b_ih[H:2 * H].reshape(1, H),
                      b_ih[2 * H:].reshape(1, H))
    bhr, bhz, bhn = (b_hh[:H].reshape(1, H), b_hh[H:2 * H].reshape(1, H),
                     b_hh[2 * H:].reshape(1, H))
    bg = b_gcn.reshape(1, H)
    gma = gamma.reshape(1, H)
    bta = beta.reshape(1, H)
    wfc16 = jnp.zeros((16, H), _f32).at[:W_fc.shape[0]].set(W_fc)
    bfc16 = jnp.zeros((1, 16), _f32).at[0, :W_fc.shape[0]].set(b_fc)

    q0, q1 = _deg_call(dst)

    h0, y0, y1, y2, dis = pl.pallas_call(
        _init_body,
        grid=(GRID,),
        in_specs=[_bs((BN, 1)), _full(emb), _bs((BN, 16)), _bs((BN, 16)),
                  _full(W_gcn)],
        out_specs=[_bs((BN, H)), _bs((BN, HG)), _bs((BN, HG)), _bs((BN, HG)),
                   _bs((BN, 1))],
        out_shape=[jax.ShapeDtypeStruct((N, H), _f32)] +
                  [jax.ShapeDtypeStruct((N, HG), _f32)] * 3 +
                  [jax.ShapeDtypeStruct((N, 1), _f32)],
    )(x2, emb, q0, q1, W_gcn)

    upd = pl.pallas_call(
        _upd_body,
        grid=(GRID,),
        in_specs=[_bs((BN, HG))] * 6 + [_bs((BN, HG))] * 3 +
                 [_bs((BN, H)), _bs((BN, 1))] +
                 [_full(w) for w in (W_gcn, wir, wiz, win, whr, whz, whn,
                                     bg, bir, biz, bin_, bhr, bhz, bhn,
                                     gma, bta)],
        out_specs=[_bs((BN, H))] + [_bs((BN, HG))] * 3,
        out_shape=[jax.ShapeDtypeStruct((N, H), _f32)] +
                  [jax.ShapeDtypeStruct((N, HG), _f32)] * 3,
    )

    def step(_, carry):
        h, a0, a1, a2 = carry
        p00, p01, p02, p10, p11, p12 = _agg_call(a0, a1, a2, src, dst)
        hn, n0, n1, n2 = upd(p00, p01, p02, p10, p11, p12, a0, a1, a2,
                             h, dis, W_gcn, wir, wiz, win, whr, whz, whn,
                             bg, bir, biz, bin_, bhr, bhz, bhn, gma, bta)
        return hn, n0, n1, n2

    h, y0, y1, y2 = lax.fori_loop(0, NUM_STEPS, step, (h0, y0, y1, y2))

    out16 = pl.pallas_call(
        _fin_body,
        grid=(GRID,),
        in_specs=[_bs((BN, H)), _full(wfc16), _full(bfc16)],
        out_specs=_bs((BN, 16)),
        out_shape=jax.ShapeDtypeStruct((N, 16), _f32),
    )(h, wfc16, bfc16)
    return out16[:, :W_fc.shape[0]]


# speedup vs baseline: 6.9387x; 1.8900x over previous
import jax
import jax.numpy as jnp
from jax import lax
from jax.experimental import pallas as pl
from jax.experimental.pallas import tpu as pltpu
from jax.experimental.pallas import tpu_sc as plsc

N = 50000
E = 800000
H = 96
HG = 32
NG = 3
NUM_STEPS = 32
VOCAB = 10
CHUNK = 128
NCHUNKS = E // CHUNK
NTILES = 32
ROWS_PAD = 50176
TROWS = ROWS_PAD // 16
ZROWS = 98
BN = 1000
GRID = N // BN

_MESH = plsc.VectorSubcoreMesh(
    core_axis_name="c", subcore_axis_name="s", num_cores=2, num_subcores=16)

_SC_PARAMS = pltpu.CompilerParams(use_tc_tiling_on_sc=False)

_f32 = jnp.float32


def _zero_vmem_2d(ref, rows, cols):
    z = jnp.zeros((16,), _f32)

    def body(i, carry):
        for k in range(cols // 16):
            ref[i, pl.ds(k * 16, 16)] = z
        return carry

    lax.fori_loop(0, rows, body, 0)


def _deg_body(dst_hbm, q0, q1, dstv, ones_b, zb, acc, _):
    c = lax.axis_index("c")
    s = lax.axis_index("s")
    w = c * 16 + s

    one = jnp.full((16,), 1.0, _f32)

    def fill(i, carry):
        ones_b[i, pl.ds(0, 16)] = one
        return carry

    lax.fori_loop(0, CHUNK, fill, 0)
    _zero_vmem_2d(zb, ZROWS, 16)

    for k in range(TROWS // ZROWS):
        pltpu.sync_copy(zb, acc.at[pl.ds(s * TROWS + k * ZROWS, ZROWS)])
    plsc.subcore_barrier()

    nch = (NCHUNKS - w + NTILES - 1) // NTILES

    def chunk(j, carry):
        base = (w + j * NTILES) * CHUNK
        pltpu.sync_copy(dst_hbm.at[pl.ds(base, CHUNK)], dstv)
        pltpu.sync_copy(ones_b, acc.at[dstv], add=True)
        return carry

    lax.fori_loop(0, nch, chunk, 0)
    plsc.subcore_barrier()

    @pl.when(c == 0)
    def _():
        pltpu.sync_copy(acc.at[pl.ds(s * TROWS, TROWS)],
                        q0.at[pl.ds(s * TROWS, TROWS)])

    @pl.when(c == 1)
    def _():
        pltpu.sync_copy(acc.at[pl.ds(s * TROWS, TROWS)],
                        q1.at[pl.ds(s * TROWS, TROWS)])


_deg_call = pl.kernel(
    _deg_body,
    out_type=(jax.ShapeDtypeStruct((ROWS_PAD, 16), _f32),
              jax.ShapeDtypeStruct((ROWS_PAD, 16), _f32)),
    mesh=_MESH,
    scratch_types=[
        pltpu.VMEM((CHUNK,), jnp.int32),
        pltpu.VMEM((CHUNK, 16), _f32),
        pltpu.VMEM((ZROWS, 16), _f32),
        pltpu.VMEM_SHARED((ROWS_PAD, 16), _f32),
        pltpu.SemaphoreType.DMA,
    ],
    compiler_params=_SC_PARAMS,
)


CPT = 196
ECHUNKS = NTILES * CPT
E_PAD = ECHUNKS * CHUNK
DUMMY_DST = N + 64
NBUF = 4
SUB = 28
NSUB = CPT // SUB


def _agg_body(y0, y1, y2, src_hbm, dst_hbm,
              p00, p01, p02, p10, p11, p12,
              sidx, didx, gb0, gb1, gb2, gb3, zb, acc,
              sg0, sg1, sg2, sg3, szero, semi):
    c = lax.axis_index("c")
    s = lax.axis_index("s")
    w = c * 16 + s

    gbufs = (gb0, gb1, gb2, gb3)
    gsems = (sg0, sg1, sg2, sg3)

    _zero_vmem_2d(zb, ZROWS, HG)

    ys = (y0, y1, y2)
    outs0 = (p00, p01, p02)
    outs1 = (p10, p11, p12)

    for g in range(NG):
        yg = ys[g]

        for k in range(TROWS // ZROWS):
            pltpu.async_copy(
                zb, acc.at[pl.ds(s * TROWS + k * ZROWS, ZROWS)], szero)
        for k in range(TROWS // ZROWS):
            pltpu.make_async_copy(
                zb, acc.at[pl.ds(s * TROWS + k * ZROWS, ZROWS)], szero).wait()
        plsc.subcore_barrier()

        for t in range(NSUB):
            da = pltpu.async_copy(
                src_hbm.at[pl.ds(w * CPT + t * SUB, SUB)], sidx, semi)
            db = pltpu.async_copy(
                dst_hbm.at[pl.ds(w * CPT + t * SUB, SUB)], didx, semi)
            da.wait()
            db.wait()

            for b in range(NBUF):
                pltpu.async_copy(yg.at[sidx.at[b]], gbufs[b], gsems[b])

            def inner(k, carry):
                for b in range(NBUF):
                    j = k * NBUF + b
                    pltpu.make_async_copy(
                        yg.at[sidx.at[j]], gbufs[b], gsems[b]).wait()
                    pltpu.sync_copy(gbufs[b], acc.at[didx.at[j]], add=True)

                    @pl.when(j + NBUF < SUB)
                    def _():
                        pltpu.async_copy(
                            yg.at[sidx.at[j + NBUF]], gbufs[b], gsems[b])
                return carry

            lax.fori_loop(0, SUB // NBUF, inner, 0)

        plsc.subcore_barrier()

        @pl.when(c == 0)
        def _():
            pltpu.sync_copy(acc.at[pl.ds(s * TROWS, TROWS)],
                            outs0[g].at[pl.ds(s * TROWS, TROWS)])

        @pl.when(c == 1)
        def _():
            pltpu.sync_copy(acc.at[pl.ds(s * TROWS, TROWS)],
                            outs1[g].at[pl.ds(s * TROWS, TROWS)])

        plsc.subcore_barrier()


_agg_call = pl.kernel(
    _agg_body,
    out_type=tuple(jax.ShapeDtypeStruct((ROWS_PAD, HG), _f32)
                   for _ in range(6)),
    mesh=_MESH,
    scratch_types=[
        pltpu.VMEM((SUB, CHUNK), jnp.int32),
        pltpu.VMEM((SUB, CHUNK), jnp.int32),
        pltpu.VMEM((CHUNK, HG), _f32),
        pltpu.VMEM((CHUNK, HG), _f32),
        pltpu.VMEM((CHUNK, HG), _f32),
        pltpu.VMEM((CHUNK, HG), _f32),
        pltpu.VMEM((ZROWS, HG), _f32),
        pltpu.VMEM_SHARED((ROWS_PAD, HG), _f32),
        pltpu.SemaphoreType.DMA,
        pltpu.SemaphoreType.DMA,
        pltpu.SemaphoreType.DMA,
        pltpu.SemaphoreType.DMA,
        pltpu.SemaphoreType.DMA,
        pltpu.SemaphoreType.DMA,
    ],
    compiler_params=_SC_PARAMS,
)


def _mm_t(a, b):
    return lax.dot_general(a.astype(jnp.bfloat16), b.astype(jnp.bfloat16),
                           (((1,), (1,)), ((), ())),
                           preferred_element_type=_f32)


def _init_body(x_ref, emb_ref, q0_ref, q1_ref, wg_ref,
               h_ref, y0_ref, y1_ref, y2_ref, dis_ref):
    deg = q0_ref[:, 0:1] + q1_ref[:, 0:1] + 1.0
    dis = lax.rsqrt(deg)
    xi = x_ref[:, :]
    h0 = jnp.zeros((xi.shape[0], H), _f32)
    for v in range(VOCAB):
        h0 = h0 + jnp.where(xi == v, 1.0, 0.0) * emb_ref[v:v + 1, :]
    y = dis * _mm_t(h0, wg_ref[:, :])
    h_ref[:, :] = h0
    y0_ref[:, :] = y[:, 0:HG]
    y1_ref[:, :] = y[:, HG:2 * HG]
    y2_ref[:, :] = y[:, 2 * HG:H]
    dis_ref[:, :] = dis


def _upd_body(p00, p01, p02, p10, p11, p12, y0, y1, y2, h_ref, dis_ref,
              wg, wir, wiz, win, whr, whz, whn,
              bg, bir, biz, bin_, bhr, bhz, bhn, gma, bta,
              hn_ref, z0_ref, z1_ref, z2_ref):
    ssum = jnp.concatenate(
        [p00[:, :] + p10[:, :], p01[:, :] + p11[:, :], p02[:, :] + p12[:, :]],
        axis=1)
    y = jnp.concatenate([y0[:, :], y1[:, :], y2[:, :]], axis=1)
    d = dis_ref[:, :]
    m = d * (ssum + y) + bg[:, :]
    m = jnp.maximum(m, 0.0)
    mu = jnp.mean(m, axis=1, keepdims=True)
    va = jnp.mean((m - mu) ** 2, axis=1, keepdims=True)
    mh = (m - mu) * lax.rsqrt(va + 1e-5) * gma[:, :] + bta[:, :]
    hh = h_ref[:, :]
    ir = _mm_t(mh, wir[:, :]) + bir[:, :]
    iz = _mm_t(mh, wiz[:, :]) + biz[:, :]
    inn = _mm_t(mh, win[:, :]) + bin_[:, :]
    hr = _mm_t(hh, whr[:, :]) + bhr[:, :]
    hz = _mm_t(hh, whz[:, :]) + bhz[:, :]
    hn = _mm_t(hh, whn[:, :]) + bhn[:, :]
    r = jax.nn.sigmoid(ir + hr)
    z = jax.nn.sigmoid(iz + hz)
    n = jnp.tanh(inn + r * hn)
    hnew = (1.0 - z) * n + z * hh
    hn_ref[:, :] = hnew
    yn = d * _mm_t(hnew, wg[:, :])
    z0_ref[:, :] = yn[:, 0:HG]
    z1_ref[:, :] = yn[:, HG:2 * HG]
    z2_ref[:, :] = yn[:, 2 * HG:H]


def _fin_body(h_ref, wfc_ref, bfc_ref, o_ref):
    o_ref[:, :] = _mm_t(h_ref[:, :], wfc_ref[:, :]) + bfc_ref[:, :]


def _bs(shape):
    return pl.BlockSpec(shape, lambda i: (i, 0))


def _full(a):
    return pl.BlockSpec(a.shape, lambda i: (0, 0))


def kernel(x, edge_index, emb, W_gcn, b_gcn, gamma, beta,
           W_ih, W_hh, b_ih, b_hh, W_fc, b_fc):
    src = edge_index[0].astype(jnp.int32)
    dst = edge_index[1].astype(jnp.int32)
    x2 = x.astype(jnp.int32).reshape(N, 1)
    src_p = jnp.concatenate(
        [src, jnp.zeros((E_PAD - E,), jnp.int32)]).reshape(ECHUNKS, CHUNK)
    dst_p = jnp.concatenate(
        [dst, jnp.full((E_PAD - E,), DUMMY_DST, jnp.int32)]
    ).reshape(ECHUNKS, CHUNK)

    wir, wiz, win = W_ih[:H], W_ih[H:2 * H], W_ih[2 * H:]
    whr, whz, whn = W_hh[:H], W_hh[H:2 * H], W_hh[2 * H:]
    bir, biz, bin_ = (b_ih[:H].reshape(1, H), b_ih[H:2 * H].reshape(1, H),
                      b_ih[2 * H:].reshape(1, H))
    bhr, bhz, bhn = (b_hh[:H].reshape(1, H), b_hh[H:2 * H].reshape(1, H),
                     b_hh[2 * H:].reshape(1, H))
    bg = b_gcn.reshape(1, H)
    gma = gamma.reshape(1, H)
    bta = beta.reshape(1, H)
    wfc16 = jnp.zeros((16, H), _f32).at[:W_fc.shape[0]].set(W_fc)
    bfc16 = jnp.zeros((1, 16), _f32).at[0, :W_fc.shape[0]].set(b_fc)

    q0, q1 = _deg_call(dst)

    h0, y0, y1, y2, dis = pl.pallas_call(
        _init_body,
        grid=(GRID,),
        in_specs=[_bs((BN, 1)), _full(emb), _bs((BN, 16)), _bs((BN, 16)),
                  _full(W_gcn)],
        out_specs=[_bs((BN, H)), _bs((BN, HG)), _bs((BN, HG)), _bs((BN, HG)),
                   _bs((BN, 1))],
        out_shape=[jax.ShapeDtypeStruct((N, H), _f32)] +
                  [jax.ShapeDtypeStruct((N, HG), _f32)] * 3 +
                  [jax.ShapeDtypeStruct((N, 1), _f32)],
    )(x2, emb, q0, q1, W_gcn)

    upd = pl.pallas_call(
        _upd_body,
        grid=(GRID,),
        in_specs=[_bs((BN, HG))] * 6 + [_bs((BN, HG))] * 3 +
                 [_bs((BN, H)), _bs((BN, 1))] +
                 [_full(w) for w in (W_gcn, wir, wiz, win, whr, whz, whn,
                                     bg, bir, biz, bin_, bhr, bhz, bhn,
                                     gma, bta)],
        out_specs=[_bs((BN, H))] + [_bs((BN, HG))] * 3,
        out_shape=[jax.ShapeDtypeStruct((N, H), _f32)] +
                  [jax.ShapeDtypeStruct((N, HG), _f32)] * 3,
    )

    def step(_, carry):
        h, a0, a1, a2 = carry
        p00, p01, p02, p10, p11, p12 = _agg_call(a0, a1, a2, src_p, dst_p)
        hn, n0, n1, n2 = upd(p00, p01, p02, p10, p11, p12, a0, a1, a2,
                             h, dis, W_gcn, wir, wiz, win, whr, whz, whn,
                             bg, bir, biz, bin_, bhr, bhz, bhn, gma, bta)
        return hn, n0, n1, n2

    h, y0, y1, y2 = lax.fori_loop(0, NUM_STEPS, step, (h0, y0, y1, y2))

    out16 = pl.pallas_call(
        _fin_body,
        grid=(GRID,),
        in_specs=[_bs((BN, H)), _full(wfc16), _full(bfc16)],
        out_specs=_bs((BN, 16)),
        out_shape=jax.ShapeDtypeStruct((N, 16), _f32),
    )(h, wfc16, bfc16)
    return out16[:, :W_fc.shape[0]]
